# baseline (speedup 1.0000x reference)
import jax
import jax.numpy as jnp
from jax import lax
from jax.experimental import pallas as pl
from jax.experimental.pallas import tpu as pltpu, tpu_sc as plsc
from jax.experimental import layout as jlayout

_B = 16384
_NW = 32
_BPW = _B // _NW
_CHUNK = 128
_NCHUNK = _BPW // _CHUNK
_MAIN = 999936
_NROWS = 1000000


def _sc_body(x_hbm, tab_hbm, tail_hbm, out_hbm,
             idx_v, idxa_v, idxb_v, idxt_v, da_v, db_v, dt_v, out_v, sem):
    nc = 2
    wid = lax.axis_index("s") * nc + lax.axis_index("c")
    base = wid * _BPW

    for k in range(_NCHUNK):
        pltpu.sync_copy(x_hbm.at[pl.ds(base + k * _CHUNK, _CHUNK)], idx_v.at[k])
    for k in range(_NCHUNK):
        for i in range(_CHUNK // 16):
            xv = idx_v.at[k][pl.ds(i * 16, 16)]
            is_tail = xv >= _MAIN
            ma = jnp.where(is_tail, 0, xv + (xv & ~127))
            idxa_v.at[k][pl.ds(i * 16, 16)] = ma
            idxb_v.at[k][pl.ds(i * 16, 16)] = ma + 128
            idxt_v.at[k][pl.ds(i * 16, 16)] = jnp.where(is_tail, xv - _MAIN, 0)

    copies = []
    for k in range(_NCHUNK):
        copies.append(pltpu.async_copy(
            tab_hbm.at[idxa_v.at[k]], da_v.at[pl.ds(k * _CHUNK, _CHUNK)], sem))
        copies.append(pltpu.async_copy(
            tab_hbm.at[idxb_v.at[k]], db_v.at[pl.ds(k * _CHUNK, _CHUNK)], sem))
        copies.append(pltpu.async_copy(
            tail_hbm.at[idxt_v.at[k]], dt_v.at[pl.ds(k * _CHUNK, _CHUNK)], sem))
    for c in copies:
        c.wait()

    for t in range(_NCHUNK):
        for i in range(_CHUNK // 16):
            o = t * _CHUNK + i * 16
            xv = idx_v.at[t][pl.ds(i * 16, 16)]
            is_tail = xv >= _MAIN
            vt = dt_v[pl.ds(o, 16)]
            d1 = jnp.where(is_tail, vt & 15, da_v[pl.ds(o, 16)])
            d2 = jnp.where(is_tail, vt >> 4, db_v[pl.ds(o, 16)])
            pw = d1 | (d2 << 4)
            for j in range(8):
                s = 3 - j if j < 4 else 11 - j
                out_v[pl.ds(t * 1024 + j * 128 + i * 16, 16)] = (
                    ((pw >> s) & 1).astype(jnp.float32))

    pltpu.sync_copy(out_v, out_hbm.at[pl.ds(base * 8, _BPW * 8)])


def kernel(x, ground_truth):
    main = ground_truth[:_MAIN]
    t = jnp.transpose(main.reshape(_MAIN // 128, 128, 2), (0, 2, 1))
    t = jlayout.with_layout_constraint(t, jlayout.Layout((0, 1, 2)))
    flat = t.reshape(-1)
    tail = ground_truth[_MAIN:]
    tailp = tail[:, 0] | (tail[:, 1] << 4)

    mesh = plsc.VectorSubcoreMesh(core_axis_name="c", subcore_axis_name="s",
                                  num_cores=2, num_subcores=16)
    out_flat = pl.kernel(
        _sc_body,
        out_type=jax.ShapeDtypeStruct((_B * 8,), jnp.float32),
        mesh=mesh,
        scratch_types=[
            pltpu.VMEM((_NCHUNK, _CHUNK), jnp.int32),
            pltpu.VMEM((_NCHUNK, _CHUNK), jnp.int32),
            pltpu.VMEM((_NCHUNK, _CHUNK), jnp.int32),
            pltpu.VMEM((_NCHUNK, _CHUNK), jnp.int32),
            pltpu.VMEM((_BPW,), jnp.int32),
            pltpu.VMEM((_BPW,), jnp.int32),
            pltpu.VMEM((_BPW,), jnp.int32),
            pltpu.VMEM((_BPW * 8,), jnp.float32),
            pltpu.SemaphoreType.DMA,
        ],
    )(x, flat, tailp)

    o3 = out_flat.reshape(_B // 128, 8, 128)
    o3 = jlayout.with_layout_constraint(o3, jlayout.Layout((0, 1, 2)))
    return jnp.transpose(o3, (0, 2, 1)).reshape(_B, 8)


# speedup vs baseline: 3.3456x; 3.3456x over previous
import jax
import jax.numpy as jnp
from jax import lax
from jax.experimental import pallas as pl
from jax.experimental.pallas import tpu as pltpu, tpu_sc as plsc
from jax.experimental import layout as jlayout

_B = 16384
_NW = 32
_BPW = _B // _NW
_CHUNK = 128
_NCHUNK = _BPW // _CHUNK
_MAIN = 999936
_NROWS = 1000000


def _sc_body(x_hbm, tab_hbm, tail_hbm, out_hbm,
             idx_v, idxa_v, idxb_v, idxt_v, da_v, db_v, dt_v, out_v, sem):
    nc = 2
    wid = lax.axis_index("s") * nc + lax.axis_index("c")
    base = wid * _BPW

    for k in range(_NCHUNK):
        pltpu.sync_copy(x_hbm.at[pl.ds(base + k * _CHUNK, _CHUNK)], idx_v.at[k])
    for k in range(_NCHUNK):
        for i in range(_CHUNK // 16):
            xv = idx_v.at[k][pl.ds(i * 16, 16)]
            is_tail = xv >= _MAIN
            ma = jnp.where(is_tail, 0, xv + (xv & ~127))
            idxa_v.at[k][pl.ds(i * 16, 16)] = ma
            idxb_v.at[k][pl.ds(i * 16, 16)] = ma + 128
            idxt_v.at[k][pl.ds(i * 16, 16)] = jnp.where(
                is_tail, xv - _MAIN, xv & 4095)

    copies = []
    for k in range(_NCHUNK):
        copies.append(pltpu.async_copy(
            tab_hbm.at[idxa_v.at[k]], da_v.at[pl.ds(k * _CHUNK, _CHUNK)], sem))
        copies.append(pltpu.async_copy(
            tab_hbm.at[idxb_v.at[k]], db_v.at[pl.ds(k * _CHUNK, _CHUNK)], sem))
        copies.append(pltpu.async_copy(
            tail_hbm.at[idxt_v.at[k]], dt_v.at[pl.ds(k * _CHUNK, _CHUNK)], sem))
    for c in copies:
        c.wait()

    for t in range(_NCHUNK):
        for i in range(_CHUNK // 16):
            o = t * _CHUNK + i * 16
            xv = idx_v.at[t][pl.ds(i * 16, 16)]
            is_tail = xv >= _MAIN
            vt = dt_v[pl.ds(o, 16)]
            d1 = jnp.where(is_tail, vt & 15, da_v[pl.ds(o, 16)])
            d2 = jnp.where(is_tail, vt >> 4, db_v[pl.ds(o, 16)])
            pw = d1 | (d2 << 4)
            for j in range(8):
                s = 3 - j if j < 4 else 11 - j
                out_v[pl.ds(t * 1024 + j * 128 + i * 16, 16)] = (
                    ((pw >> s) & 1).astype(jnp.float32))

    pltpu.sync_copy(out_v, out_hbm.at[pl.ds(base * 8, _BPW * 8)])


def kernel(x, ground_truth):
    main = ground_truth[:_MAIN]
    t = jnp.transpose(main.reshape(_MAIN // 128, 128, 2), (0, 2, 1))
    t = jlayout.with_layout_constraint(t, jlayout.Layout((0, 1, 2)))
    flat = t.reshape(-1)
    tail = ground_truth[_MAIN:]
    tailp = jnp.pad(tail[:, 0] | (tail[:, 1] << 4), (0, 4096 - (_NROWS - _MAIN)))

    mesh = plsc.VectorSubcoreMesh(core_axis_name="c", subcore_axis_name="s",
                                  num_cores=2, num_subcores=16)
    out_flat = pl.kernel(
        _sc_body,
        out_type=jax.ShapeDtypeStruct((_B * 8,), jnp.float32),
        mesh=mesh,
        scratch_types=[
            pltpu.VMEM((_NCHUNK, _CHUNK), jnp.int32),
            pltpu.VMEM((_NCHUNK, _CHUNK), jnp.int32),
            pltpu.VMEM((_NCHUNK, _CHUNK), jnp.int32),
            pltpu.VMEM((_NCHUNK, _CHUNK), jnp.int32),
            pltpu.VMEM((_BPW,), jnp.int32),
            pltpu.VMEM((_BPW,), jnp.int32),
            pltpu.VMEM((_BPW,), jnp.int32),
            pltpu.VMEM((_BPW * 8,), jnp.float32),
            pltpu.SemaphoreType.DMA,
        ],
    )(x, flat, tailp)

    o3 = out_flat.reshape(_B // 128, 8, 128)
    o3 = jlayout.with_layout_constraint(o3, jlayout.Layout((0, 1, 2)))
    return jnp.transpose(o3, (0, 2, 1)).reshape(_B, 8)


# speedup vs baseline: 3.4959x; 1.0449x over previous
import jax
import jax.numpy as jnp
from jax import lax
from jax.experimental import pallas as pl
from jax.experimental.pallas import tpu as pltpu, tpu_sc as plsc
from jax.experimental import layout as jlayout

_B = 16384
_NW = 32
_BPW = _B // _NW
_CHUNK = 128
_NCHUNK = _BPW // _CHUNK
_MAIN = 999936
_NROWS = 1000000
_TAILPAD = 4096


def _sc_body(x_hbm, tab_hbm, tail_hbm, out_hbm,
             idx_v, idxa_v, idxb_v, idxt_v, da_v, db_v, dt_v, out_v, sem):
    nc = 2
    wid = lax.axis_index("s") * nc + lax.axis_index("c")
    base = wid * _BPW

    pltpu.sync_copy(x_hbm.at[pl.ds(base, _BPW)], idx_v)

    def build(i, carry):
        xv = idx_v[pl.ds(i * 16, 16)]
        is_tail = xv >= _MAIN
        ma = jnp.where(is_tail, 0, xv + (xv & ~127))
        idxa_v[pl.ds(i * 16, 16)] = ma
        idxb_v[pl.ds(i * 16, 16)] = ma + 128
        idxt_v[pl.ds(i * 16, 16)] = jnp.where(is_tail, xv - _MAIN,
                                              xv & (_TAILPAD - 1))
        return carry

    lax.fori_loop(0, _BPW // 16, build, 0, unroll=2)

    copies = []
    for k in range(_NCHUNK):
        s = pl.ds(k * _CHUNK, _CHUNK)
        copies.append(pltpu.async_copy(tab_hbm.at[idxa_v.at[s]], da_v.at[s], sem))
        copies.append(pltpu.async_copy(tab_hbm.at[idxb_v.at[s]], db_v.at[s], sem))
        copies.append(pltpu.async_copy(tail_hbm.at[idxt_v.at[s]], dt_v.at[s], sem))
    for c in copies:
        c.wait()

    def unpack(i, carry):
        t = i // 8
        r = i % 8
        o = i * 16
        xv = idx_v[pl.ds(o, 16)]
        is_tail = xv >= _MAIN
        vt = dt_v[pl.ds(o, 16)]
        d1 = jnp.where(is_tail, vt & 15, da_v[pl.ds(o, 16)])
        d2 = jnp.where(is_tail, vt >> 4, db_v[pl.ds(o, 16)])
        pw = d1 | (d2 << 4)
        for j in range(8):
            s = 3 - j if j < 4 else 11 - j
            out_v[pl.ds(t * 1024 + j * 128 + r * 16, 16)] = (
                ((pw >> s) & 1).astype(jnp.float32))
        return carry

    lax.fori_loop(0, _BPW // 16, unpack, 0)

    pltpu.sync_copy(out_v, out_hbm.at[pl.ds(base * 8, _BPW * 8)])


def kernel(x, ground_truth):
    main = ground_truth[:_MAIN]
    t = jnp.transpose(main.reshape(_MAIN // 128, 128, 2), (0, 2, 1))
    t = jlayout.with_layout_constraint(t, jlayout.Layout((0, 1, 2)))
    flat = t.reshape(-1)
    tail = ground_truth[_MAIN:]
    tailp = jnp.pad(tail[:, 0] | (tail[:, 1] << 4),
                    (0, _TAILPAD - (_NROWS - _MAIN)))

    mesh = plsc.VectorSubcoreMesh(core_axis_name="c", subcore_axis_name="s",
                                  num_cores=2, num_subcores=16)
    out_flat = pl.kernel(
        _sc_body,
        out_type=jax.ShapeDtypeStruct((_B * 8,), jnp.float32),
        mesh=mesh,
        scratch_types=[
            pltpu.VMEM((_BPW,), jnp.int32),
            pltpu.VMEM((_BPW,), jnp.int32),
            pltpu.VMEM((_BPW,), jnp.int32),
            pltpu.VMEM((_BPW,), jnp.int32),
            pltpu.VMEM((_BPW,), jnp.int32),
            pltpu.VMEM((_BPW,), jnp.int32),
            pltpu.VMEM((_BPW,), jnp.int32),
            pltpu.VMEM((_BPW * 8,), jnp.float32),
            pltpu.SemaphoreType.DMA,
        ],
    )(x, flat, tailp)

    o3 = out_flat.reshape(_B // 128, 8, 128)
    o3 = jlayout.with_layout_constraint(o3, jlayout.Layout((0, 1, 2)))
    return jnp.transpose(o3, (0, 2, 1)).reshape(_B, 8)
